# baseline (speedup 1.0000x reference)
import functools

import jax
import jax.numpy as jnp
from jax import lax
from jax.experimental import pallas as pl
from jax.experimental.pallas import tpu as pltpu
from jax.experimental.pallas import tpu_sc as plsc

N = 10000
D = 128
E = 320000
NC = 2
NS = 16
NW = NC * NS
EB = 128
NB = 80
HNB = 40
KBUF = 2
C0B = 160
C0PH = ((0, 40), (40, 40), (80, 40), (120, 40))
EPW = NB * EB
EP = EPW * NW
NP = 10240
RPT = NP // NS
R15 = 15 * RPT
NCHUNK = NP // D

_mesh = plsc.VectorSubcoreMesh(core_axis_name="c", subcore_axis_name="s")


@functools.partial(
    pl.kernel,
    out_type=jax.ShapeDtypeStruct((NC, NP), jnp.float32),
    mesh=_mesh,
    compiler_params=pltpu.CompilerParams(needs_layout_passes=False),
    scratch_types=[
        pltpu.VMEM((EPW,), jnp.int32),
        pltpu.VMEM((NP,), jnp.float32),
        pltpu.VMEM((NS, RPT), jnp.float32),
        pltpu.VMEM((RPT,), jnp.float32),
        pltpu.VMEM_SHARED((NS, NP), jnp.float32),
    ],
)
def _deg_kernel(dst_hbm, zeros_hbm, out_hbm, idx_v, hist, slab, comb, stage):
    c = lax.axis_index("c")
    s = lax.axis_index("s")
    pltpu.sync_copy(zeros_hbm, hist)
    base = pl.multiple_of((c * NS + s) * EPW, 8)
    pltpu.sync_copy(dst_hbm.at[pl.ds(base, EPW)], idx_v)
    ones = jnp.ones((16,), jnp.float32)

    def inner(k, carry2):
        idx = idx_v[pl.ds(k * 16, 16)]
        plsc.addupdate_scatter(hist, [idx], ones)
        return carry2

    lax.fori_loop(0, EPW // 16, inner, 0)
    pltpu.sync_copy(hist, stage.at[s])
    plsc.subcore_barrier()
    r0 = pl.multiple_of(s * RPT, 8)
    pltpu.sync_copy(stage.at[:, pl.ds(r0, RPT)], slab)

    def comb_body(j, carry):
        v = slab[0, pl.ds(j * 16, 16)]
        for h in range(1, NS):
            v = v + slab[h, pl.ds(j * 16, 16)]
        comb[pl.ds(j * 16, 16)] = v
        return carry

    lax.fori_loop(0, RPT // 16, comb_body, 0)
    pltpu.sync_copy(comb, out_hbm.at[c, pl.ds(r0, RPT)])


@functools.partial(
    pl.kernel,
    out_type=jax.ShapeDtypeStruct((NP, D), jnp.float32),
    mesh=_mesh,
    scratch_types=[
        pltpu.VMEM((HNB, EB), jnp.int32),
        pltpu.VMEM((HNB, EB), jnp.int32),
        [pltpu.VMEM((EB, D), jnp.float32) for _ in range(KBUF)],
        [pltpu.SemaphoreType.DMA for _ in range(KBUF)],
        pltpu.VMEM_SHARED((NP, D), jnp.float32),
    ],
)
def _seg_kernel(src_hbm, dst_hbm, y_hbm, zeros_hbm, out_hbm, si, di, rows, sems, acc):
    c = lax.axis_index("c")
    s = lax.axis_index("s")
    r0 = pl.multiple_of(s * RPT, 8)

    @pl.when(c == 0)
    def _():
        @pl.when(s < NS - 1)
        def _():
            pltpu.sync_copy(y_hbm.at[pl.ds(r0, RPT)], acc.at[pl.ds(r0, RPT)])

        @pl.when(s == NS - 1)
        def _():
            pltpu.sync_copy(y_hbm.at[pl.ds(R15, N - R15)], acc.at[pl.ds(R15, N - R15)])
            pltpu.sync_copy(zeros_hbm.at[pl.ds(N, NP - N)], acc.at[pl.ds(N, NP - N)])

    plsc.subcore_barrier()

    def pipeline(base_block, phases):
        for off, nb in phases:
            hb = pl.multiple_of(base_block + off, 8)
            pltpu.sync_copy(src_hbm.at[pl.ds(hb, nb)], si.at[pl.ds(0, nb)])
            pltpu.sync_copy(dst_hbm.at[pl.ds(hb, nb)], di.at[pl.ds(0, nb)])
            for b in range(KBUF):
                pltpu.async_copy(y_hbm.at[si.at[b]], rows[b], sems[b])

            def group(g, carry):
                for b in range(KBUF):
                    j = g * KBUF + b
                    pltpu.make_async_copy(y_hbm.at[si.at[j]], rows[b], sems[b]).wait()
                    pltpu.sync_copy(rows[b], acc.at[di.at[j]], add=True)
                    jn = j + KBUF

                    @pl.when(jn < nb)
                    def _():
                        pltpu.async_copy(y_hbm.at[si.at[jn]], rows[b], sems[b])
                return carry

            lax.fori_loop(0, nb // KBUF, group, 0)

    @pl.when(c == 0)
    def _():
        pipeline(pl.multiple_of(s * C0B, 8), C0PH)

    plsc.subcore_barrier()

    @pl.when(c == 0)
    def _():
        pltpu.sync_copy(acc.at[pl.ds(r0, RPT)], out_hbm.at[pl.ds(r0, RPT)])


def _dis_body(dega_ref, degb_ref, disb_ref):
    deg = 1.0 + dega_ref[0, 0, :] + degb_ref[0, 0, :]
    dis = lax.rsqrt(deg)
    disb_ref[...] = jnp.broadcast_to(dis[None, :], (D, D)).T


def _dense1_body(disb_ref, emb_ref, w1_ref, y1_ref):
    xw = jnp.dot(emb_ref[...], w1_ref[...], preferred_element_type=jnp.float32)
    y1_ref[...] = disb_ref[0:N, :] * xw


def _dense2_body(s1_ref, disb_ref, b1_ref, w2_ref, y2_ref):
    dis = disb_ref[0:N, :]
    x2 = jnp.maximum(dis * s1_ref[0:N, :] + b1_ref[...], 0.0)
    y2_ref[...] = dis * jnp.dot(x2, w2_ref[...], preferred_element_type=jnp.float32)


def _dense3_body(s2_ref, disb_ref, b2_ref, out_ref):
    out_ref[...] = disb_ref[0:N, :] * s2_ref[0:N, :] + b2_ref[...]


_f32 = jnp.float32
_nd = jax.ShapeDtypeStruct((N, D), _f32)

_dis_call = pl.pallas_call(
    _dis_body,
    grid=(NCHUNK,),
    in_specs=[
        pl.BlockSpec((1, 1, D), lambda r: (r, 0, 0)),
        pl.BlockSpec((1, 1, D), lambda r: (r, 0, 0)),
    ],
    out_specs=pl.BlockSpec((D, D), lambda r: (r, 0)),
    out_shape=jax.ShapeDtypeStruct((NP, D), _f32),
)


def kernel(prop_edge_index, emb_weight, W1, b1, W2, b2):
    src = prop_edge_index[0].astype(jnp.int32)
    dst = prop_edge_index[1].astype(jnp.int32)
    pad = EP - E
    src_p = jnp.concatenate([src, jnp.zeros((pad,), jnp.int32)])
    dst_p = jnp.concatenate([dst, jnp.full((pad,), N, jnp.int32)])
    src_2d = src_p.reshape(EP // EB, EB)
    dst_2d = dst_p.reshape(EP // EB, EB)
    zeros1 = jnp.zeros((NP,), _f32)
    zeros_nd = jnp.zeros((NP, D), _f32)

    deg2 = _deg_kernel(dst_p, zeros1)
    dega3 = deg2[0].reshape(NCHUNK, 1, D)
    degb3 = deg2[1].reshape(NCHUNK, 1, D)
    disb = _dis_call(dega3, degb3)

    y1 = pl.pallas_call(_dense1_body, out_shape=_nd)(disb, emb_weight, W1)
    s1 = _seg_kernel(src_2d, dst_2d, y1, zeros_nd)
    y2 = pl.pallas_call(_dense2_body, out_shape=_nd)(
        s1, disb, b1.reshape(1, D), W2)
    s2 = _seg_kernel(src_2d, dst_2d, y2, zeros_nd)
    out = pl.pallas_call(_dense3_body, out_shape=_nd)(
        s2, disb, b2.reshape(1, D))
    return out


# speedup vs baseline: 1.1565x; 1.1565x over previous
import functools

import jax
import jax.numpy as jnp
from jax import lax
from jax.experimental import pallas as pl
from jax.experimental.pallas import tpu as pltpu
from jax.experimental.pallas import tpu_sc as plsc

N = 10000
D = 128
E = 320000
NC = 2
NS = 16
NW = NC * NS
EB = 128
NB = 80
HNB = 40
KBUF = 2
C0B = 120
C1B = 40
C0PH = ((0, 40), (40, 40), (80, 40))
EPW = NB * EB
EP = EPW * NW
NP = 10240
RPT = NP // NS
R15 = 15 * RPT
NCHUNK = NP // D

_mesh = plsc.VectorSubcoreMesh(core_axis_name="c", subcore_axis_name="s")


@functools.partial(
    pl.kernel,
    out_type=jax.ShapeDtypeStruct((NC, NP), jnp.float32),
    mesh=_mesh,
    compiler_params=pltpu.CompilerParams(needs_layout_passes=False),
    scratch_types=[
        pltpu.VMEM((EPW,), jnp.int32),
        pltpu.VMEM((NP,), jnp.float32),
        pltpu.VMEM((NS, RPT), jnp.float32),
        pltpu.VMEM((RPT,), jnp.float32),
        pltpu.VMEM_SHARED((NS, NP), jnp.float32),
    ],
)
def _deg_kernel(dst_hbm, zeros_hbm, out_hbm, idx_v, hist, slab, comb, stage):
    c = lax.axis_index("c")
    s = lax.axis_index("s")
    pltpu.sync_copy(zeros_hbm, hist)
    base = pl.multiple_of((c * NS + s) * EPW, 8)
    pltpu.sync_copy(dst_hbm.at[pl.ds(base, EPW)], idx_v)
    ones = jnp.ones((16,), jnp.float32)

    def inner(k, carry2):
        idx = idx_v[pl.ds(k * 16, 16)]
        plsc.addupdate_scatter(hist, [idx], ones)
        return carry2

    lax.fori_loop(0, EPW // 16, inner, 0)
    pltpu.sync_copy(hist, stage.at[s])
    plsc.subcore_barrier()
    r0 = pl.multiple_of(s * RPT, 8)
    pltpu.sync_copy(stage.at[:, pl.ds(r0, RPT)], slab)

    def comb_body(j, carry):
        v = slab[0, pl.ds(j * 16, 16)]
        for h in range(1, NS):
            v = v + slab[h, pl.ds(j * 16, 16)]
        comb[pl.ds(j * 16, 16)] = v
        return carry

    lax.fori_loop(0, RPT // 16, comb_body, 0)
    pltpu.sync_copy(comb, out_hbm.at[c, pl.ds(r0, RPT)])


@functools.partial(
    pl.kernel,
    out_type=jax.ShapeDtypeStruct((NC, NP, D), jnp.float32),
    mesh=_mesh,
    scratch_types=[
        pltpu.VMEM((HNB, EB), jnp.int32),
        pltpu.VMEM((HNB, EB), jnp.int32),
        [pltpu.VMEM((EB, D), jnp.float32) for _ in range(KBUF)],
        [pltpu.SemaphoreType.DMA for _ in range(KBUF)],
        pltpu.VMEM_SHARED((NP, D), jnp.float32),
    ],
)
def _seg_kernel(src_hbm, dst_hbm, y_hbm, zeros_hbm, out_hbm, si, di, rows, sems, acc):
    c = lax.axis_index("c")
    s = lax.axis_index("s")
    r0 = pl.multiple_of(s * RPT, 8)

    @pl.when(c == 0)
    def _():
        @pl.when(s < NS - 1)
        def _():
            pltpu.sync_copy(y_hbm.at[pl.ds(r0, RPT)], acc.at[pl.ds(r0, RPT)])

        @pl.when(s == NS - 1)
        def _():
            pltpu.sync_copy(y_hbm.at[pl.ds(R15, N - R15)], acc.at[pl.ds(R15, N - R15)])
            pltpu.sync_copy(zeros_hbm.at[pl.ds(N, NP - N)], acc.at[pl.ds(N, NP - N)])

    @pl.when(c != 0)
    def _():
        pltpu.sync_copy(zeros_hbm.at[pl.ds(r0, RPT)], acc.at[pl.ds(r0, RPT)])

    plsc.subcore_barrier()

    def pipeline(base_block, phases):
        for off, nb in phases:
            hb = pl.multiple_of(base_block + off, 8)
            pltpu.sync_copy(src_hbm.at[pl.ds(hb, nb)], si.at[pl.ds(0, nb)])
            pltpu.sync_copy(dst_hbm.at[pl.ds(hb, nb)], di.at[pl.ds(0, nb)])
            for b in range(KBUF):
                pltpu.async_copy(y_hbm.at[si.at[b]], rows[b], sems[b])

            def group(g, carry):
                for b in range(KBUF):
                    j = g * KBUF + b
                    pltpu.make_async_copy(y_hbm.at[si.at[j]], rows[b], sems[b]).wait()
                    pltpu.sync_copy(rows[b], acc.at[di.at[j]], add=True)
                    jn = j + KBUF

                    @pl.when(jn < nb)
                    def _():
                        pltpu.async_copy(y_hbm.at[si.at[jn]], rows[b], sems[b])
                return carry

            lax.fori_loop(0, nb // KBUF, group, 0)

    @pl.when(c == 0)
    def _():
        pipeline(pl.multiple_of(s * C0B, 8), C0PH)

    @pl.when(c != 0)
    def _():
        hb = pl.multiple_of(NS * C0B + s * C1B, 8)
        pltpu.sync_copy(src_hbm.at[pl.ds(hb, C1B)], si.at[pl.ds(0, C1B)])
        pltpu.sync_copy(dst_hbm.at[pl.ds(hb, C1B)], di.at[pl.ds(0, C1B)])

        def serial(j, carry):
            pltpu.async_copy(y_hbm.at[si.at[j]], rows[0], sems[0]).wait()
            pltpu.sync_copy(rows[0], acc.at[di.at[j]], add=True)
            return carry

        lax.fori_loop(0, C1B, serial, 0)

    plsc.subcore_barrier()
    pltpu.sync_copy(acc.at[pl.ds(r0, RPT)], out_hbm.at[c, pl.ds(r0, RPT)])


def _dis_body(dega_ref, degb_ref, disb_ref):
    deg = 1.0 + dega_ref[0, 0, :] + degb_ref[0, 0, :]
    dis = lax.rsqrt(deg)
    disb_ref[...] = jnp.broadcast_to(dis[None, :], (D, D)).T


def _dense1_body(disb_ref, emb_ref, w1_ref, y1_ref):
    xw = jnp.dot(emb_ref[...], w1_ref[...], preferred_element_type=jnp.float32)
    y1_ref[...] = disb_ref[0:N, :] * xw


def _dense2_body(s1a_ref, s1b_ref, disb_ref, b1_ref, w2_ref, y2_ref):
    dis = disb_ref[0:N, :]
    x2 = jnp.maximum(dis * (s1a_ref[0:N, :] + s1b_ref[0:N, :]) + b1_ref[...], 0.0)
    y2_ref[...] = dis * jnp.dot(x2, w2_ref[...], preferred_element_type=jnp.float32)


def _dense3_body(s2a_ref, s2b_ref, disb_ref, b2_ref, out_ref):
    out_ref[...] = (disb_ref[0:N, :] * (s2a_ref[0:N, :] + s2b_ref[0:N, :])
                    + b2_ref[...])


_f32 = jnp.float32
_nd = jax.ShapeDtypeStruct((N, D), _f32)

_dis_call = pl.pallas_call(
    _dis_body,
    grid=(NCHUNK,),
    in_specs=[
        pl.BlockSpec((1, 1, D), lambda r: (r, 0, 0)),
        pl.BlockSpec((1, 1, D), lambda r: (r, 0, 0)),
    ],
    out_specs=pl.BlockSpec((D, D), lambda r: (r, 0)),
    out_shape=jax.ShapeDtypeStruct((NP, D), _f32),
)


def kernel(prop_edge_index, emb_weight, W1, b1, W2, b2):
    src = prop_edge_index[0].astype(jnp.int32)
    dst = prop_edge_index[1].astype(jnp.int32)
    pad = EP - E
    src_p = jnp.concatenate([src, jnp.zeros((pad,), jnp.int32)])
    dst_p = jnp.concatenate([dst, jnp.full((pad,), N, jnp.int32)])
    src_2d = src_p.reshape(EP // EB, EB)
    dst_2d = dst_p.reshape(EP // EB, EB)
    zeros1 = jnp.zeros((NP,), _f32)
    zeros_nd = jnp.zeros((NP, D), _f32)

    deg2 = _deg_kernel(dst_p, zeros1)
    dega3 = deg2[0].reshape(NCHUNK, 1, D)
    degb3 = deg2[1].reshape(NCHUNK, 1, D)
    disb = _dis_call(dega3, degb3)

    y1 = pl.pallas_call(_dense1_body, out_shape=_nd)(disb, emb_weight, W1)
    s1 = _seg_kernel(src_2d, dst_2d, y1, zeros_nd)
    y2 = pl.pallas_call(_dense2_body, out_shape=_nd)(
        s1[0], s1[1], disb, b1.reshape(1, D), W2)
    s2 = _seg_kernel(src_2d, dst_2d, y2, zeros_nd)
    out = pl.pallas_call(_dense3_body, out_shape=_nd)(
        s2[0], s2[1], disb, b2.reshape(1, D))
    return out


# speedup vs baseline: 1.4004x; 1.2108x over previous
import functools

import jax
import jax.numpy as jnp
from jax import lax
from jax.experimental import pallas as pl
from jax.experimental.pallas import tpu as pltpu
from jax.experimental.pallas import tpu_sc as plsc

N = 10000
D = 128
E = 320000
NC = 2
NS = 16
NW = NC * NS
EB = 128
NB0 = 96
NB1 = 62
EPB = NS * (NB0 + NB1)
EP = EPB * EB
NB = 79
EPW = NB * EB
NP = 10240
RPT = NP // NS
R15 = 15 * RPT
NCHUNK = NP // D

_mesh = plsc.VectorSubcoreMesh(core_axis_name="c", subcore_axis_name="s")


@functools.partial(
    pl.kernel,
    out_type=jax.ShapeDtypeStruct((NC, NP), jnp.float32),
    mesh=_mesh,
    compiler_params=pltpu.CompilerParams(needs_layout_passes=False),
    scratch_types=[
        pltpu.VMEM((EPW,), jnp.int32),
        pltpu.VMEM((NP,), jnp.float32),
        pltpu.VMEM((NS, RPT), jnp.float32),
        pltpu.VMEM((RPT,), jnp.float32),
        pltpu.VMEM_SHARED((NS, NP), jnp.float32),
    ],
)
def _deg_kernel(dst_hbm, zeros_hbm, out_hbm, idx_v, hist, slab, comb, stage):
    c = lax.axis_index("c")
    s = lax.axis_index("s")
    pltpu.sync_copy(zeros_hbm, hist)
    base = pl.multiple_of((c * NS + s) * EPW, 8)
    pltpu.sync_copy(dst_hbm.at[pl.ds(base, EPW)], idx_v)
    ones = jnp.ones((16,), jnp.float32)

    def inner(k, carry2):
        idx = idx_v[pl.ds(k * 16, 16)]
        plsc.addupdate_scatter(hist, [idx], ones)
        return carry2

    lax.fori_loop(0, EPW // 16, inner, 0)
    pltpu.sync_copy(hist, stage.at[s])
    plsc.subcore_barrier()
    r0 = pl.multiple_of(s * RPT, 8)
    pltpu.sync_copy(stage.at[:, pl.ds(r0, RPT)], slab)

    def comb_body(j, carry):
        v = slab[0, pl.ds(j * 16, 16)]
        for h in range(1, NS):
            v = v + slab[h, pl.ds(j * 16, 16)]
        comb[pl.ds(j * 16, 16)] = v
        return carry

    lax.fori_loop(0, RPT // 16, comb_body, 0)
    pltpu.sync_copy(comb, out_hbm.at[c, pl.ds(r0, RPT)])


@functools.partial(
    pl.kernel,
    out_type=jax.ShapeDtypeStruct((NC, NP, D), jnp.float32),
    mesh=_mesh,
    scratch_types=[
        pltpu.VMEM((EB,), jnp.int32),
        pltpu.VMEM((EB,), jnp.int32),
        pltpu.VMEM((EB, D), jnp.float32),
        pltpu.VMEM_SHARED((NP, D), jnp.float32),
        pltpu.SemaphoreType.DMA,
    ],
)
def _seg_kernel(src_hbm, dst_hbm, y_hbm, zeros_hbm, out_hbm, si, di, rows, acc, sem):
    c = lax.axis_index("c")
    s = lax.axis_index("s")
    r0 = pl.multiple_of(s * RPT, 8)

    @pl.when(c == 0)
    def _():
        @pl.when(s < NS - 1)
        def _():
            pltpu.sync_copy(y_hbm.at[pl.ds(r0, RPT)], acc.at[pl.ds(r0, RPT)])

        @pl.when(s == NS - 1)
        def _():
            pltpu.sync_copy(y_hbm.at[pl.ds(R15, N - R15)], acc.at[pl.ds(R15, N - R15)])
            pltpu.sync_copy(zeros_hbm.at[pl.ds(N, NP - N)], acc.at[pl.ds(N, NP - N)])

    @pl.when(c != 0)
    def _():
        pltpu.sync_copy(zeros_hbm.at[pl.ds(r0, RPT)], acc.at[pl.ds(r0, RPT)])

    plsc.subcore_barrier()

    def run(base_edge, nblocks):
        def body(j, carry):
            off = pl.multiple_of(base_edge + j * EB, EB)
            pltpu.sync_copy(src_hbm.at[pl.ds(off, EB)], si)
            pltpu.sync_copy(dst_hbm.at[pl.ds(off, EB)], di)
            pltpu.async_copy(y_hbm.at[si], rows, sem).wait()
            pltpu.sync_copy(rows, acc.at[di], add=True)
            return carry

        lax.fori_loop(0, nblocks, body, 0)

    @pl.when(c == 0)
    def _():
        run(pl.multiple_of(s * NB0 * EB, EB), NB0)

    @pl.when(c != 0)
    def _():
        run(pl.multiple_of((NS * NB0 + s * NB1) * EB, EB), NB1)

    plsc.subcore_barrier()
    pltpu.sync_copy(acc.at[pl.ds(r0, RPT)], out_hbm.at[c, pl.ds(r0, RPT)])


def _dis_body(dega_ref, degb_ref, disb_ref):
    deg = 1.0 + dega_ref[0, 0, :] + degb_ref[0, 0, :]
    dis = lax.rsqrt(deg)
    disb_ref[...] = jnp.broadcast_to(dis[None, :], (D, D)).T


def _dense1_body(disb_ref, emb_ref, w1_ref, y1_ref):
    xw = jnp.dot(emb_ref[...], w1_ref[...], preferred_element_type=jnp.float32)
    y1_ref[...] = disb_ref[0:N, :] * xw


def _dense2_body(s1a_ref, s1b_ref, disb_ref, b1_ref, w2_ref, y2_ref):
    dis = disb_ref[0:N, :]
    x2 = jnp.maximum(dis * (s1a_ref[0:N, :] + s1b_ref[0:N, :]) + b1_ref[...], 0.0)
    y2_ref[...] = dis * jnp.dot(x2, w2_ref[...], preferred_element_type=jnp.float32)


def _dense3_body(s2a_ref, s2b_ref, disb_ref, b2_ref, out_ref):
    out_ref[...] = (disb_ref[0:N, :] * (s2a_ref[0:N, :] + s2b_ref[0:N, :])
                    + b2_ref[...])


_f32 = jnp.float32
_nd = jax.ShapeDtypeStruct((N, D), _f32)

_dis_call = pl.pallas_call(
    _dis_body,
    grid=(NCHUNK,),
    in_specs=[
        pl.BlockSpec((1, 1, D), lambda r: (r, 0, 0)),
        pl.BlockSpec((1, 1, D), lambda r: (r, 0, 0)),
    ],
    out_specs=pl.BlockSpec((D, D), lambda r: (r, 0)),
    out_shape=jax.ShapeDtypeStruct((NP, D), _f32),
)


def kernel(prop_edge_index, emb_weight, W1, b1, W2, b2):
    src = prop_edge_index[0].astype(jnp.int32)
    dst = prop_edge_index[1].astype(jnp.int32)
    pad = EP - E
    src_p = jnp.concatenate([src, jnp.zeros((pad,), jnp.int32)])
    dst_p = jnp.concatenate([dst, jnp.full((pad,), N, jnp.int32)])
    zeros1 = jnp.zeros((NP,), _f32)
    zeros_nd = jnp.zeros((NP, D), _f32)

    deg2 = _deg_kernel(dst_p, zeros1)
    dega3 = deg2[0].reshape(NCHUNK, 1, D)
    degb3 = deg2[1].reshape(NCHUNK, 1, D)
    disb = _dis_call(dega3, degb3)

    y1 = pl.pallas_call(_dense1_body, out_shape=_nd)(disb, emb_weight, W1)
    s1 = _seg_kernel(src_p, dst_p, y1, zeros_nd)
    y2 = pl.pallas_call(_dense2_body, out_shape=_nd)(
        s1[0], s1[1], disb, b1.reshape(1, D), W2)
    s2 = _seg_kernel(src_p, dst_p, y2, zeros_nd)
    out = pl.pallas_call(_dense3_body, out_shape=_nd)(
        s2[0], s2[1], disb, b2.reshape(1, D))
    return out


# speedup vs baseline: 1.4403x; 1.0285x over previous
import functools

import jax
import jax.numpy as jnp
from jax import lax
from jax.experimental import pallas as pl
from jax.experimental.pallas import tpu as pltpu
from jax.experimental.pallas import tpu_sc as plsc

N = 10000
D = 128
E = 320000
NC = 2
NS = 16
NW = NC * NS
EB = 128
NB0 = 104
NB1 = 54
EPB = NS * (NB0 + NB1)
EP = EPB * EB
NB = 79
EPW = NB * EB
NP = 10240
RPT = NP // NS
R15 = 15 * RPT
NCHUNK = NP // D

_mesh = plsc.VectorSubcoreMesh(core_axis_name="c", subcore_axis_name="s")


@functools.partial(
    pl.kernel,
    out_type=jax.ShapeDtypeStruct((NC, NP), jnp.float32),
    mesh=_mesh,
    compiler_params=pltpu.CompilerParams(needs_layout_passes=False),
    scratch_types=[
        pltpu.VMEM((EPW,), jnp.int32),
        pltpu.VMEM((NP,), jnp.float32),
        pltpu.VMEM((NS, RPT), jnp.float32),
        pltpu.VMEM((RPT,), jnp.float32),
        pltpu.VMEM_SHARED((NS, NP), jnp.float32),
    ],
)
def _deg_kernel(dst_hbm, zeros_hbm, out_hbm, idx_v, hist, slab, comb, stage):
    c = lax.axis_index("c")
    s = lax.axis_index("s")
    pltpu.sync_copy(zeros_hbm, hist)
    base = pl.multiple_of((c * NS + s) * EPW, 8)
    pltpu.sync_copy(dst_hbm.at[pl.ds(base, EPW)], idx_v)
    ones = jnp.ones((16,), jnp.float32)

    def inner(k, carry2):
        idx = idx_v[pl.ds(k * 16, 16)]
        plsc.addupdate_scatter(hist, [idx], ones)
        return carry2

    lax.fori_loop(0, EPW // 16, inner, 0)
    pltpu.sync_copy(hist, stage.at[s])
    plsc.subcore_barrier()
    r0 = pl.multiple_of(s * RPT, 8)
    pltpu.sync_copy(stage.at[:, pl.ds(r0, RPT)], slab)

    def comb_body(j, carry):
        v = slab[0, pl.ds(j * 16, 16)]
        for h in range(1, NS):
            v = v + slab[h, pl.ds(j * 16, 16)]
        comb[pl.ds(j * 16, 16)] = v
        return carry

    lax.fori_loop(0, RPT // 16, comb_body, 0)
    pltpu.sync_copy(comb, out_hbm.at[c, pl.ds(r0, RPT)])


@functools.partial(
    pl.kernel,
    out_type=jax.ShapeDtypeStruct((NC, NP, D), jnp.float32),
    mesh=_mesh,
    scratch_types=[
        pltpu.VMEM((EB,), jnp.int32),
        pltpu.VMEM((EB,), jnp.int32),
        pltpu.VMEM((EB, D), jnp.float32),
        pltpu.VMEM_SHARED((NP, D), jnp.float32),
        pltpu.SemaphoreType.DMA,
    ],
)
def _seg_kernel(src_hbm, dst_hbm, y_hbm, zeros_hbm, out_hbm, si, di, rows, acc, sem):
    c = lax.axis_index("c")
    s = lax.axis_index("s")
    r0 = pl.multiple_of(s * RPT, 8)

    @pl.when(c == 0)
    def _():
        @pl.when(s < NS - 1)
        def _():
            pltpu.sync_copy(y_hbm.at[pl.ds(r0, RPT)], acc.at[pl.ds(r0, RPT)])

        @pl.when(s == NS - 1)
        def _():
            pltpu.sync_copy(y_hbm.at[pl.ds(R15, N - R15)], acc.at[pl.ds(R15, N - R15)])
            pltpu.sync_copy(zeros_hbm.at[pl.ds(N, NP - N)], acc.at[pl.ds(N, NP - N)])

    @pl.when(c != 0)
    def _():
        pltpu.sync_copy(zeros_hbm.at[pl.ds(r0, RPT)], acc.at[pl.ds(r0, RPT)])

    plsc.subcore_barrier()

    def run(base_edge, nblocks):
        def body(j, carry):
            off = pl.multiple_of(base_edge + j * EB, EB)
            pltpu.sync_copy(src_hbm.at[pl.ds(off, EB)], si)
            pltpu.sync_copy(dst_hbm.at[pl.ds(off, EB)], di)
            pltpu.async_copy(y_hbm.at[si], rows, sem).wait()
            pltpu.sync_copy(rows, acc.at[di], add=True)
            return carry

        lax.fori_loop(0, nblocks, body, 0)

    @pl.when(c == 0)
    def _():
        run(pl.multiple_of(s * NB0 * EB, EB), NB0)

    @pl.when(c != 0)
    def _():
        run(pl.multiple_of((NS * NB0 + s * NB1) * EB, EB), NB1)

    plsc.subcore_barrier()
    pltpu.sync_copy(acc.at[pl.ds(r0, RPT)], out_hbm.at[c, pl.ds(r0, RPT)])


def _dis_body(dega_ref, degb_ref, disb_ref):
    deg = 1.0 + dega_ref[0, 0, :] + degb_ref[0, 0, :]
    dis = lax.rsqrt(deg)
    disb_ref[...] = jnp.broadcast_to(dis[None, :], (D, D)).T


def _dense1_body(disb_ref, emb_ref, w1_ref, y1_ref):
    xw = jnp.dot(emb_ref[...], w1_ref[...], preferred_element_type=jnp.float32)
    y1_ref[...] = disb_ref[0:N, :] * xw


def _dense2_body(s1a_ref, s1b_ref, disb_ref, b1_ref, w2_ref, y2_ref):
    dis = disb_ref[0:N, :]
    x2 = jnp.maximum(dis * (s1a_ref[0:N, :] + s1b_ref[0:N, :]) + b1_ref[...], 0.0)
    y2_ref[...] = dis * jnp.dot(x2, w2_ref[...], preferred_element_type=jnp.float32)


def _dense3_body(s2a_ref, s2b_ref, disb_ref, b2_ref, out_ref):
    out_ref[...] = (disb_ref[0:N, :] * (s2a_ref[0:N, :] + s2b_ref[0:N, :])
                    + b2_ref[...])


_f32 = jnp.float32
_nd = jax.ShapeDtypeStruct((N, D), _f32)

_dis_call = pl.pallas_call(
    _dis_body,
    grid=(NCHUNK,),
    in_specs=[
        pl.BlockSpec((1, 1, D), lambda r: (r, 0, 0)),
        pl.BlockSpec((1, 1, D), lambda r: (r, 0, 0)),
    ],
    out_specs=pl.BlockSpec((D, D), lambda r: (r, 0)),
    out_shape=jax.ShapeDtypeStruct((NP, D), _f32),
)


def kernel(prop_edge_index, emb_weight, W1, b1, W2, b2):
    src = prop_edge_index[0].astype(jnp.int32)
    dst = prop_edge_index[1].astype(jnp.int32)
    pad = EP - E
    src_p = jnp.concatenate([src, jnp.zeros((pad,), jnp.int32)])
    dst_p = jnp.concatenate([dst, jnp.full((pad,), N, jnp.int32)])
    zeros1 = jnp.zeros((NP,), _f32)
    zeros_nd = jnp.zeros((NP, D), _f32)

    deg2 = _deg_kernel(dst_p, zeros1)
    dega3 = deg2[0].reshape(NCHUNK, 1, D)
    degb3 = deg2[1].reshape(NCHUNK, 1, D)
    disb = _dis_call(dega3, degb3)

    y1 = pl.pallas_call(_dense1_body, out_shape=_nd)(disb, emb_weight, W1)
    s1 = _seg_kernel(src_p, dst_p, y1, zeros_nd)
    y2 = pl.pallas_call(_dense2_body, out_shape=_nd)(
        s1[0], s1[1], disb, b1.reshape(1, D), W2)
    s2 = _seg_kernel(src_p, dst_p, y2, zeros_nd)
    out = pl.pallas_call(_dense3_body, out_shape=_nd)(
        s2[0], s2[1], disb, b2.reshape(1, D))
    return out
